# baseline (speedup 1.0000x reference)
import functools

import jax
import jax.numpy as jnp
from jax import lax
from jax.experimental import pallas as pl
from jax.experimental.pallas import tpu as pltpu
from jax.experimental.pallas import tpu_sc as plsc

N_NODE = 5000
NP = 5120
H = 128
E = 160000
EP = 163840
ER = EP // 128
OUT_DIM = 256
NC, NS = 2, 16
NW = NC * NS
GPW = NP // NW
ER_T = ER // NS
RPT = NP // NS
RB = 4
BLK = NP // RB

f32 = jnp.float32


def _fill128(ref, rows, value):
  @pl.loop(0, rows)
  def _(i):
    @pl.loop(0, 8)
    def _(j):
      ref[i, pl.ds(j * 16, 16)] = jnp.full((16,), value, f32)


def _sc_front_body(diag_tid, desc_tid, d2s_src, d2s_dst, s2d_src, s2d_dst,
                   diag_table, desc_table, ones_hbm, zer_hbm,
                   hd_out, hs_out, dg_od_d2s, dg_id_d2s, dg_od_s2d, dg_id_s2d,
                   idx_v, rows_v, ei_v, ones_v, hv_v, hist_od, hist_id, sem):
  cid = lax.axis_index("c")
  sid = lax.axis_index("s")
  wid = sid * NC + cid

  pltpu.sync_copy(ones_hbm, ones_v)
  pltpu.sync_copy(zer_hbm, hv_v)
  pltpu.sync_copy(hv_v, hist_od.at[pl.ds(sid * RPT, RPT)])
  pltpu.sync_copy(hv_v, hist_id.at[pl.ds(sid * RPT, RPT)])
  plsc.subcore_barrier()

  for tbl, tid, out in ((diag_table, diag_tid, hd_out),
                        (desc_table, desc_tid, hs_out)):
    @pl.loop(0, GPW // 80)
    def _(t):
      base = wid * GPW + t * 80
      pltpu.sync_copy(tid.at[pl.ds(base, 80)], idx_v)
      pltpu.async_copy(tbl.at[idx_v], rows_v, sem).wait()
      pltpu.sync_copy(rows_v, out.at[pl.ds(base, 80)])

  def hist_pass(src2, dst2):
    for e2, hist in ((src2, hist_od), (dst2, hist_id)):
      pltpu.sync_copy(e2.at[pl.ds(sid * ER_T, ER_T)], ei_v)
      @pl.loop(0, ER_T)
      def _(j):
        pltpu.sync_copy(ones_v, hist.at[ei_v.at[j]], add=True)

  @pl.when(cid == 0)
  def _():
    hist_pass(d2s_src, d2s_dst)

  @pl.when(cid == 1)
  def _():
    hist_pass(s2d_src, s2d_dst)

  plsc.subcore_barrier()

  def hist_out(out_od, out_id):
    for hist, out in ((hist_od, out_od), (hist_id, out_id)):
      pltpu.sync_copy(hist.at[pl.ds(sid * RPT, RPT)], hv_v)
      pltpu.sync_copy(hv_v, out.at[pl.ds(sid * RPT, RPT)])

  @pl.when(cid == 0)
  def _():
    hist_out(dg_od_d2s, dg_id_d2s)

  @pl.when(cid == 1)
  def _():
    hist_out(dg_od_s2d, dg_id_s2d)


_sc_front = functools.partial(
    pl.kernel,
    out_type=[jax.ShapeDtypeStruct((NP, H), f32),
              jax.ShapeDtypeStruct((NP, H), f32),
              jax.ShapeDtypeStruct((NP,), f32),
              jax.ShapeDtypeStruct((NP,), f32),
              jax.ShapeDtypeStruct((NP,), f32),
              jax.ShapeDtypeStruct((NP,), f32)],
    mesh=plsc.VectorSubcoreMesh(core_axis_name="c", subcore_axis_name="s",
                                num_cores=NC, num_subcores=NS),
    scratch_types=[pltpu.VMEM((80,), jnp.int32),
                   pltpu.VMEM((80, H), f32),
                   pltpu.VMEM((ER_T, 128), jnp.int32),
                   pltpu.VMEM((128,), f32),
                   pltpu.VMEM((RPT,), f32),
                   pltpu.VMEM_SHARED((NP,), f32),
                   pltpu.VMEM_SHARED((NP,), f32),
                   pltpu.SemaphoreType.DMA],
)(_sc_front_body)


NB = 4


def _sc_segsum_body(m_diag, m_desc, d2s_src, d2s_dst, s2d_src, s2d_dst,
                    agg_desc, agg_diag,
                    isrc_v, idst_v, bufs, z_v, acc, gsem, ssem):
  cid = lax.axis_index("c")
  sid = lax.axis_index("s")

  _fill128(z_v, 16, 0.0)
  @pl.loop(0, RPT // 16)
  def _(t):
    pltpu.sync_copy(z_v, acc.at[pl.ds(sid * RPT + t * 16, 16)])
  plsc.subcore_barrier()

  def edge_pass(m, esrc, edst):
    pltpu.sync_copy(esrc.at[pl.ds(sid * ER_T, ER_T)], isrc_v)
    pltpu.sync_copy(edst.at[pl.ds(sid * ER_T, ER_T)], idst_v)

    def g(j, b):
      return pltpu.make_async_copy(m.at[isrc_v.at[j]], bufs.at[b],
                                   gsem.at[b])

    def s(j, b):
      return pltpu.make_async_copy(bufs.at[b], acc.at[idst_v.at[j]],
                                   ssem.at[b])

    for b in range(NB):
      g(b, b).start()

    @pl.loop(0, ER_T // NB - 1)
    def _(j4):
      j = j4 * NB
      for b in range(NB):
        g(j + b, b).wait()
        s(j + b, b).start(add=True)
      for b in range(NB):
        s(j + b, b).wait()
        g(j + NB + b, b).start()

    jl = ER_T - NB
    for b in range(NB):
      g(jl + b, b).wait()
      s(jl + b, b).start(add=True)
    for b in range(NB):
      s(jl + b, b).wait()

  @pl.when(cid == 0)
  def _():
    edge_pass(m_diag, d2s_src, d2s_dst)

  @pl.when(cid == 1)
  def _():
    edge_pass(m_desc, s2d_src, s2d_dst)

  plsc.subcore_barrier()

  def readout(out):
    @pl.loop(0, RPT // 16)
    def _(t):
      pltpu.sync_copy(acc.at[pl.ds(sid * RPT + t * 16, 16)], z_v)
      pltpu.sync_copy(z_v, out.at[pl.ds(sid * RPT + t * 16, 16)])

  @pl.when(cid == 0)
  def _():
    readout(agg_desc)

  @pl.when(cid == 1)
  def _():
    readout(agg_diag)


_sc_segsum = functools.partial(
    pl.kernel,
    out_type=[jax.ShapeDtypeStruct((NP, H), f32),
              jax.ShapeDtypeStruct((NP, H), f32)],
    mesh=plsc.VectorSubcoreMesh(core_axis_name="c", subcore_axis_name="s",
                                num_cores=NC, num_subcores=NS),
    scratch_types=[pltpu.VMEM((ER_T, 128), jnp.int32),
                   pltpu.VMEM((ER_T, 128), jnp.int32),
                   pltpu.VMEM((NB, 128, H), f32),
                   pltpu.VMEM((16, H), f32),
                   pltpu.VMEM_SHARED((NP, H), f32),
                   pltpu.SemaphoreType.DMA((NB,)),
                   pltpu.SemaphoreType.DMA((NB,))],
)(_sc_segsum_body)


def _rs(x):
  return lax.rsqrt(jnp.maximum(x, 1.0))


def _ln_scale_body(hd, hs, g, b, odd, ods, md, ms):
  def ln(x):
    mu = jnp.mean(x, axis=-1, keepdims=True)
    var = jnp.mean((x - mu) ** 2, axis=-1, keepdims=True)
    y = (x - mu) * lax.rsqrt(var + 1e-5) * g[...] + b[...]
    return jnp.nan_to_num(y, nan=0.0, posinf=0.0, neginf=0.0)
  md[...] = ln(hd[...]) * _rs(odd[...])
  ms[...] = ln(hs[...]) * _rs(ods[...])


def _ln_scale(hd, hs, g, b, odd, ods):
  row = pl.BlockSpec((BLK, H), lambda i: (i, 0))
  vec = pl.BlockSpec((1, H), lambda i: (0, 0))
  return pl.pallas_call(
      _ln_scale_body,
      grid=(RB,),
      in_specs=[row, row, vec, vec, row, row],
      out_specs=[row, row],
      out_shape=[jax.ShapeDtypeStruct((NP, H), f32)] * 2,
  )(hd, hs, g, b, odd, ods)


def _layer_body(aggd, aggg, idd, ids_, odd, ods, wd, bd, ws, bs, m1d, m1s):
  h1s = jnp.maximum(
      jnp.dot(aggd[...] * _rs(idd[...]), wd[...],
              preferred_element_type=f32) + bd[...], 0.0)
  m1s[...] = h1s * _rs(ods[...])
  h1d = jnp.maximum(
      jnp.dot(aggg[...] * _rs(ids_[...]), ws[...],
              preferred_element_type=f32) + bs[...], 0.0)
  m1d[...] = h1d * _rs(odd[...])


def _layer(aggd, aggg, idd, ids_, odd, ods, wd, bd, ws, bs):
  row = pl.BlockSpec((BLK, H), lambda i: (i, 0))
  mat = pl.BlockSpec((H, H), lambda i: (0, 0))
  vec = pl.BlockSpec((1, H), lambda i: (0, 0))
  return pl.pallas_call(
      _layer_body,
      grid=(RB,),
      in_specs=[row, row, row, row, row, row, mat, vec, mat, vec],
      out_specs=[row, row],
      out_shape=[jax.ShapeDtypeStruct((NP, H), f32)] * 2,
  )(aggd, aggg, idd, ids_, odd, ods, wd, bd, ws, bs)


def _pool_body(aggd, aggg, idd, ids_, wd, bd, ws, bs, sd, sg):
  i = pl.program_id(0)
  h2s = jnp.maximum(
      jnp.dot(aggd[...] * _rs(idd[...]), wd[...],
              preferred_element_type=f32) + bd[...], 0.0)
  h2d = jnp.maximum(
      jnp.dot(aggg[...] * _rs(ids_[...]), ws[...],
              preferred_element_type=f32) + bs[...], 0.0)
  mask = (lax.broadcasted_iota(jnp.int32, (BLK, H), 0) + i * BLK) < N_NODE
  sd[...] = jnp.sum(jnp.where(mask, h2s, 0.0), axis=0).reshape(1, 1, H)
  sg[...] = jnp.sum(jnp.where(mask, h2d, 0.0), axis=0).reshape(1, 1, H)


def _pool(aggd, aggg, idd, ids_, wd, bd, ws, bs):
  row = pl.BlockSpec((BLK, H), lambda i: (i, 0))
  mat = pl.BlockSpec((H, H), lambda i: (0, 0))
  vec = pl.BlockSpec((1, H), lambda i: (0, 0))
  out = pl.BlockSpec((1, 1, H), lambda i: (i, 0, 0))
  return pl.pallas_call(
      _pool_body,
      grid=(RB,),
      in_specs=[row, row, row, row, mat, vec, mat, vec],
      out_specs=[out, out],
      out_shape=[jax.ShapeDtypeStruct((RB, 1, H), f32)] * 2,
  )(aggd, aggg, idd, ids_, wd, bd, ws, bs)


def _head_body(sd, sg, pw, pb, out):
  g = (jnp.sum(sd[...], axis=0)
       + jnp.sum(sg[...], axis=0)) / float(N_NODE)
  g = jnp.nan_to_num(g, nan=0.0, posinf=0.0, neginf=0.0)
  o = jnp.dot(g, pw[...], preferred_element_type=f32) + pb[...]
  o = jnp.nan_to_num(o, nan=0.0, posinf=0.0, neginf=0.0)
  n = jnp.sqrt(jnp.sum(o * o, axis=-1, keepdims=True))
  out[...] = o / jnp.maximum(n, 1e-6)


def _head(sd, sg, pw, pb):
  return pl.pallas_call(
      _head_body,
      out_shape=jax.ShapeDtypeStruct((1, OUT_DIM), f32),
  )(sd, sg, pw, pb)


def _pad_edges(ei):
  pad = N_NODE + (jnp.arange(EP - E, dtype=jnp.int32) % (NP - N_NODE))
  src = jnp.concatenate([ei[0], pad]).reshape(ER, 128)
  dst = jnp.concatenate([ei[1], pad]).reshape(ER, 128)
  return src, dst


def kernel(diag_tid, desc_tid, ei_d2s, ei_s2d, diag_table, desc_table,
           ln_g, ln_b, W_d2s_0, b_d2s_0, W_s2d_0, b_s2d_0,
           W_d2s_1, b_d2s_1, W_s2d_1, b_s2d_1, proj_W, proj_b):
  pad_tid = jnp.arange(NP - N_NODE, dtype=jnp.int32)
  diag_tid_p = jnp.concatenate([diag_tid, pad_tid])
  desc_tid_p = jnp.concatenate([desc_tid, pad_tid])
  d2s_s, d2s_d = _pad_edges(ei_d2s)
  s2d_s, s2d_d = _pad_edges(ei_s2d)

  hd_raw, hs_raw, dg_od_d2s, dg_id_d2s, dg_od_s2d, dg_id_s2d = _sc_front(
      diag_tid_p, desc_tid_p, d2s_s, d2s_d, s2d_s, s2d_d,
      diag_table, desc_table,
      jnp.ones((128,), f32), jnp.zeros((RPT,), f32))

  def bcast(dg):
    return jnp.broadcast_to(dg[:, None], (NP, H))

  odb_d2s, idb_d2s = bcast(dg_od_d2s), bcast(dg_id_d2s)
  odb_s2d, idb_s2d = bcast(dg_od_s2d), bcast(dg_id_s2d)

  g2, b2 = ln_g.reshape(1, H), ln_b.reshape(1, H)
  m0_diag, m0_desc = _ln_scale(hd_raw, hs_raw, g2, b2, odb_d2s, odb_s2d)

  agg0_desc, agg0_diag = _sc_segsum(m0_diag, m0_desc,
                                    d2s_s, d2s_d, s2d_s, s2d_d)

  m1_diag, m1_desc = _layer(agg0_desc, agg0_diag, idb_d2s, idb_s2d,
                            odb_d2s, odb_s2d,
                            W_d2s_0, b_d2s_0.reshape(1, H),
                            W_s2d_0, b_s2d_0.reshape(1, H))

  agg1_desc, agg1_diag = _sc_segsum(m1_diag, m1_desc,
                                    d2s_s, d2s_d, s2d_s, s2d_d)

  sd, sg = _pool(agg1_desc, agg1_diag, idb_d2s, idb_s2d,
                 W_d2s_1, b_d2s_1.reshape(1, H),
                 W_s2d_1, b_s2d_1.reshape(1, H))

  return _head(sd, sg, proj_W, proj_b.reshape(1, OUT_DIM))


# speedup vs baseline: 1.0772x; 1.0772x over previous
import functools

import jax
import jax.numpy as jnp
from jax import lax
from jax.experimental import pallas as pl
from jax.experimental.pallas import tpu as pltpu
from jax.experimental.pallas import tpu_sc as plsc

N_NODE = 5000
NP = 5120
H = 128
E = 160000
EP = 163840
ER = EP // 128
OUT_DIM = 256
NC, NS = 2, 16
NW = NC * NS
GPW = NP // NW
ER_T = ER // NS
RPT = NP // NS
RB = 4
BLK = NP // RB

f32 = jnp.float32


def _fill128(ref, rows, value):
  @pl.loop(0, rows)
  def _(i):
    @pl.loop(0, 8)
    def _(j):
      ref[i, pl.ds(j * 16, 16)] = jnp.full((16,), value, f32)


def _sc_front_body(diag_tid, desc_tid, d2s_src, d2s_dst, s2d_src, s2d_dst,
                   diag_table, desc_table, ones_hbm, zer_hbm,
                   hd_out, hs_out, dg_od_d2s, dg_id_d2s, dg_od_s2d, dg_id_s2d,
                   idx_v, rows_v, ei_v, ones_v, hv_v, hist_od, hist_id, sem):
  cid = lax.axis_index("c")
  sid = lax.axis_index("s")
  wid = sid * NC + cid

  pltpu.sync_copy(ones_hbm, ones_v)
  pltpu.sync_copy(zer_hbm, hv_v)
  pltpu.sync_copy(hv_v, hist_od.at[pl.ds(sid * RPT, RPT)])
  pltpu.sync_copy(hv_v, hist_id.at[pl.ds(sid * RPT, RPT)])
  plsc.subcore_barrier()

  for tbl, tid, out in ((diag_table, diag_tid, hd_out),
                        (desc_table, desc_tid, hs_out)):
    @pl.loop(0, GPW // 80)
    def _(t):
      base = wid * GPW + t * 80
      pltpu.sync_copy(tid.at[pl.ds(base, 80)], idx_v)
      pltpu.async_copy(tbl.at[idx_v], rows_v, sem).wait()
      pltpu.sync_copy(rows_v, out.at[pl.ds(base, 80)])

  def hist_pass(src2, dst2):
    for e2, hist in ((src2, hist_od), (dst2, hist_id)):
      pltpu.sync_copy(e2.at[pl.ds(sid * ER_T, ER_T)], ei_v)
      @pl.loop(0, ER_T)
      def _(j):
        pltpu.sync_copy(ones_v, hist.at[ei_v.at[j]], add=True)

  @pl.when(cid == 0)
  def _():
    hist_pass(d2s_src, d2s_dst)

  @pl.when(cid == 1)
  def _():
    hist_pass(s2d_src, s2d_dst)

  plsc.subcore_barrier()

  def hist_out(out_od, out_id):
    for hist, out in ((hist_od, out_od), (hist_id, out_id)):
      pltpu.sync_copy(hist.at[pl.ds(sid * RPT, RPT)], hv_v)
      pltpu.sync_copy(hv_v, out.at[pl.ds(sid * RPT, RPT)])

  @pl.when(cid == 0)
  def _():
    hist_out(dg_od_d2s, dg_id_d2s)

  @pl.when(cid == 1)
  def _():
    hist_out(dg_od_s2d, dg_id_s2d)


_sc_front = functools.partial(
    pl.kernel,
    out_type=[jax.ShapeDtypeStruct((NP, H), f32),
              jax.ShapeDtypeStruct((NP, H), f32),
              jax.ShapeDtypeStruct((NP,), f32),
              jax.ShapeDtypeStruct((NP,), f32),
              jax.ShapeDtypeStruct((NP,), f32),
              jax.ShapeDtypeStruct((NP,), f32)],
    mesh=plsc.VectorSubcoreMesh(core_axis_name="c", subcore_axis_name="s",
                                num_cores=NC, num_subcores=NS),
    scratch_types=[pltpu.VMEM((80,), jnp.int32),
                   pltpu.VMEM((80, H), f32),
                   pltpu.VMEM((ER_T, 128), jnp.int32),
                   pltpu.VMEM((128,), f32),
                   pltpu.VMEM((RPT,), f32),
                   pltpu.VMEM_SHARED((NP,), f32),
                   pltpu.VMEM_SHARED((NP,), f32),
                   pltpu.SemaphoreType.DMA],
)(_sc_front_body)


NB = 4


def _sc_segsum_body(m_diag, m_desc, d2s_src, d2s_dst, s2d_src, s2d_dst,
                    agg_desc, agg_diag,
                    isrc_v, idst_v, bufs, z_v, acc, gsem, ssem):
  cid = lax.axis_index("c")
  sid = lax.axis_index("s")

  _fill128(z_v, 16, 0.0)
  @pl.loop(0, RPT // 16)
  def _(t):
    pltpu.sync_copy(z_v, acc.at[pl.ds(sid * RPT + t * 16, 16)])
  plsc.subcore_barrier()

  def edge_pass(m, esrc, edst):
    pltpu.sync_copy(esrc.at[pl.ds(sid * ER_T, ER_T)], isrc_v)
    pltpu.sync_copy(edst.at[pl.ds(sid * ER_T, ER_T)], idst_v)

    def g(j, b):
      return pltpu.make_async_copy(m.at[isrc_v.at[j]], bufs.at[b],
                                   gsem.at[b])

    def s(j, b):
      return pltpu.make_async_copy(bufs.at[b], acc.at[idst_v.at[j]],
                                   ssem.at[b])

    g(0, 0).start()
    g(1, 1).start()
    g(0, 0).wait()
    s(0, 0).start(add=True)
    g(2, 2).start()
    g(1, 1).wait()
    s(1, 1).start(add=True)
    g(3, 3).start()

    @pl.loop(0, (ER_T - NB) // NB)
    def _(j4):
      j = j4 * NB + 2
      for i in range(NB):
        jj = j + i
        b = (2 + i) % NB
        s(jj - 2, (b - 2) % NB).wait()
        g(jj + 2, (b - 2) % NB).start()
        g(jj, b).wait()
        s(jj, b).start(add=True)

    jl = ER_T - 2
    s(jl - 2, (jl - 2) % NB).wait()
    s(jl - 1, (jl - 1) % NB).wait()
    g(jl, jl % NB).wait()
    s(jl, jl % NB).start(add=True)
    g(jl + 1, (jl + 1) % NB).wait()
    s(jl + 1, (jl + 1) % NB).start(add=True)
    s(jl, jl % NB).wait()
    s(jl + 1, (jl + 1) % NB).wait()

  @pl.when(cid == 0)
  def _():
    edge_pass(m_diag, d2s_src, d2s_dst)

  @pl.when(cid == 1)
  def _():
    edge_pass(m_desc, s2d_src, s2d_dst)

  plsc.subcore_barrier()

  def readout(out):
    @pl.loop(0, RPT // 16)
    def _(t):
      pltpu.sync_copy(acc.at[pl.ds(sid * RPT + t * 16, 16)], z_v)
      pltpu.sync_copy(z_v, out.at[pl.ds(sid * RPT + t * 16, 16)])

  @pl.when(cid == 0)
  def _():
    readout(agg_desc)

  @pl.when(cid == 1)
  def _():
    readout(agg_diag)


_sc_segsum = functools.partial(
    pl.kernel,
    out_type=[jax.ShapeDtypeStruct((NP, H), f32),
              jax.ShapeDtypeStruct((NP, H), f32)],
    mesh=plsc.VectorSubcoreMesh(core_axis_name="c", subcore_axis_name="s",
                                num_cores=NC, num_subcores=NS),
    scratch_types=[pltpu.VMEM((ER_T, 128), jnp.int32),
                   pltpu.VMEM((ER_T, 128), jnp.int32),
                   pltpu.VMEM((NB, 128, H), f32),
                   pltpu.VMEM((16, H), f32),
                   pltpu.VMEM_SHARED((NP, H), f32),
                   pltpu.SemaphoreType.DMA((NB,)),
                   pltpu.SemaphoreType.DMA((NB,))],
)(_sc_segsum_body)


def _rs(x):
  return lax.rsqrt(jnp.maximum(x, 1.0))


def _ln_scale_body(hd, hs, g, b, odd, ods, md, ms):
  def ln(x):
    mu = jnp.mean(x, axis=-1, keepdims=True)
    var = jnp.mean((x - mu) ** 2, axis=-1, keepdims=True)
    y = (x - mu) * lax.rsqrt(var + 1e-5) * g[...] + b[...]
    return jnp.nan_to_num(y, nan=0.0, posinf=0.0, neginf=0.0)
  md[...] = ln(hd[...]) * _rs(odd[...])
  ms[...] = ln(hs[...]) * _rs(ods[...])


def _ln_scale(hd, hs, g, b, odd, ods):
  row = pl.BlockSpec((BLK, H), lambda i: (i, 0))
  vec = pl.BlockSpec((1, H), lambda i: (0, 0))
  return pl.pallas_call(
      _ln_scale_body,
      grid=(RB,),
      in_specs=[row, row, vec, vec, row, row],
      out_specs=[row, row],
      out_shape=[jax.ShapeDtypeStruct((NP, H), f32)] * 2,
  )(hd, hs, g, b, odd, ods)


def _layer_body(aggd, aggg, idd, ids_, odd, ods, wd, bd, ws, bs, m1d, m1s):
  h1s = jnp.maximum(
      jnp.dot(aggd[...] * _rs(idd[...]), wd[...],
              preferred_element_type=f32) + bd[...], 0.0)
  m1s[...] = h1s * _rs(ods[...])
  h1d = jnp.maximum(
      jnp.dot(aggg[...] * _rs(ids_[...]), ws[...],
              preferred_element_type=f32) + bs[...], 0.0)
  m1d[...] = h1d * _rs(odd[...])


def _layer(aggd, aggg, idd, ids_, odd, ods, wd, bd, ws, bs):
  row = pl.BlockSpec((BLK, H), lambda i: (i, 0))
  mat = pl.BlockSpec((H, H), lambda i: (0, 0))
  vec = pl.BlockSpec((1, H), lambda i: (0, 0))
  return pl.pallas_call(
      _layer_body,
      grid=(RB,),
      in_specs=[row, row, row, row, row, row, mat, vec, mat, vec],
      out_specs=[row, row],
      out_shape=[jax.ShapeDtypeStruct((NP, H), f32)] * 2,
  )(aggd, aggg, idd, ids_, odd, ods, wd, bd, ws, bs)


def _pool_body(aggd, aggg, idd, ids_, wd, bd, ws, bs, sd, sg):
  i = pl.program_id(0)
  h2s = jnp.maximum(
      jnp.dot(aggd[...] * _rs(idd[...]), wd[...],
              preferred_element_type=f32) + bd[...], 0.0)
  h2d = jnp.maximum(
      jnp.dot(aggg[...] * _rs(ids_[...]), ws[...],
              preferred_element_type=f32) + bs[...], 0.0)
  mask = (lax.broadcasted_iota(jnp.int32, (BLK, H), 0) + i * BLK) < N_NODE
  sd[...] = jnp.sum(jnp.where(mask, h2s, 0.0), axis=0).reshape(1, 1, H)
  sg[...] = jnp.sum(jnp.where(mask, h2d, 0.0), axis=0).reshape(1, 1, H)


def _pool(aggd, aggg, idd, ids_, wd, bd, ws, bs):
  row = pl.BlockSpec((BLK, H), lambda i: (i, 0))
  mat = pl.BlockSpec((H, H), lambda i: (0, 0))
  vec = pl.BlockSpec((1, H), lambda i: (0, 0))
  out = pl.BlockSpec((1, 1, H), lambda i: (i, 0, 0))
  return pl.pallas_call(
      _pool_body,
      grid=(RB,),
      in_specs=[row, row, row, row, mat, vec, mat, vec],
      out_specs=[out, out],
      out_shape=[jax.ShapeDtypeStruct((RB, 1, H), f32)] * 2,
  )(aggd, aggg, idd, ids_, wd, bd, ws, bs)


def _head_body(sd, sg, pw, pb, out):
  g = (jnp.sum(sd[...], axis=0)
       + jnp.sum(sg[...], axis=0)) / float(N_NODE)
  g = jnp.nan_to_num(g, nan=0.0, posinf=0.0, neginf=0.0)
  o = jnp.dot(g, pw[...], preferred_element_type=f32) + pb[...]
  o = jnp.nan_to_num(o, nan=0.0, posinf=0.0, neginf=0.0)
  n = jnp.sqrt(jnp.sum(o * o, axis=-1, keepdims=True))
  out[...] = o / jnp.maximum(n, 1e-6)


def _head(sd, sg, pw, pb):
  return pl.pallas_call(
      _head_body,
      out_shape=jax.ShapeDtypeStruct((1, OUT_DIM), f32),
  )(sd, sg, pw, pb)


def _pad_edges(ei):
  pad = N_NODE + (jnp.arange(EP - E, dtype=jnp.int32) % (NP - N_NODE))
  src = jnp.concatenate([ei[0], pad]).reshape(ER, 128)
  dst = jnp.concatenate([ei[1], pad]).reshape(ER, 128)
  return src, dst


def kernel(diag_tid, desc_tid, ei_d2s, ei_s2d, diag_table, desc_table,
           ln_g, ln_b, W_d2s_0, b_d2s_0, W_s2d_0, b_s2d_0,
           W_d2s_1, b_d2s_1, W_s2d_1, b_s2d_1, proj_W, proj_b):
  pad_tid = jnp.arange(NP - N_NODE, dtype=jnp.int32)
  diag_tid_p = jnp.concatenate([diag_tid, pad_tid])
  desc_tid_p = jnp.concatenate([desc_tid, pad_tid])
  d2s_s, d2s_d = _pad_edges(ei_d2s)
  s2d_s, s2d_d = _pad_edges(ei_s2d)

  hd_raw, hs_raw, dg_od_d2s, dg_id_d2s, dg_od_s2d, dg_id_s2d = _sc_front(
      diag_tid_p, desc_tid_p, d2s_s, d2s_d, s2d_s, s2d_d,
      diag_table, desc_table,
      jnp.ones((128,), f32), jnp.zeros((RPT,), f32))

  def bcast(dg):
    return jnp.broadcast_to(dg[:, None], (NP, H))

  odb_d2s, idb_d2s = bcast(dg_od_d2s), bcast(dg_id_d2s)
  odb_s2d, idb_s2d = bcast(dg_od_s2d), bcast(dg_id_s2d)

  g2, b2 = ln_g.reshape(1, H), ln_b.reshape(1, H)
  m0_diag, m0_desc = _ln_scale(hd_raw, hs_raw, g2, b2, odb_d2s, odb_s2d)

  agg0_desc, agg0_diag = _sc_segsum(m0_diag, m0_desc,
                                    d2s_s, d2s_d, s2d_s, s2d_d)

  m1_diag, m1_desc = _layer(agg0_desc, agg0_diag, idb_d2s, idb_s2d,
                            odb_d2s, odb_s2d,
                            W_d2s_0, b_d2s_0.reshape(1, H),
                            W_s2d_0, b_s2d_0.reshape(1, H))

  agg1_desc, agg1_diag = _sc_segsum(m1_diag, m1_desc,
                                    d2s_s, d2s_d, s2d_s, s2d_d)

  sd, sg = _pool(agg1_desc, agg1_diag, idb_d2s, idb_s2d,
                 W_d2s_1, b_d2s_1.reshape(1, H),
                 W_s2d_1, b_s2d_1.reshape(1, H))

  return _head(sd, sg, proj_W, proj_b.reshape(1, OUT_DIM))


# speedup vs baseline: 1.1621x; 1.0789x over previous
import functools

import jax
import jax.numpy as jnp
from jax import lax
from jax.experimental import pallas as pl
from jax.experimental.pallas import tpu as pltpu
from jax.experimental.pallas import tpu_sc as plsc

N_NODE = 5000
NP = 5120
H = 128
E = 160000
EP = 163840
ER = EP // 128
OUT_DIM = 256
NC, NS = 2, 16
NW = NC * NS
GPW = NP // NW
ER_T = ER // NS
RPT = NP // NS
RB = 4
BLK = NP // RB

f32 = jnp.float32


def _fill128(ref, rows, value):
  @pl.loop(0, rows)
  def _(i):
    @pl.loop(0, 8)
    def _(j):
      ref[i, pl.ds(j * 16, 16)] = jnp.full((16,), value, f32)


def _sc_front_body(diag_tid, desc_tid, e_d2s, e_s2d,
                   diag_table, desc_table, ones_hbm, zer_hbm,
                   hd_out, hs_out, dg_od_d2s, dg_id_d2s, dg_od_s2d, dg_id_s2d,
                   idx_v, rows_d, rows_s, eia_v, eib_v, ones_v, hv_v,
                   hist_od, hist_id, gsem, hsem):
  cid = lax.axis_index("c")
  sid = lax.axis_index("s")
  wid = sid * NC + cid

  pltpu.sync_copy(ones_hbm, ones_v)
  pltpu.sync_copy(zer_hbm, hv_v)
  pltpu.sync_copy(hv_v, hist_od.at[pl.ds(sid * RPT, RPT)])
  pltpu.sync_copy(hv_v, hist_id.at[pl.ds(sid * RPT, RPT)])

  emb = []
  for k, (tbl, tid, rows) in enumerate(((diag_table, diag_tid, rows_d),
                                        (desc_table, desc_tid, rows_s))):
    for t in range(2):
      base = wid * GPW + t * 80
      pltpu.sync_copy(tid.at[pl.ds(base, 80)], idx_v.at[2 * k + t])
      d = pltpu.make_async_copy(tbl.at[idx_v.at[2 * k + t]],
                                rows.at[pl.ds(t * 80, 80)],
                                gsem.at[2 * k + t])
      d.start()
      emb.append(d)
  plsc.subcore_barrier()

  def hist_pass(e3):
    pltpu.sync_copy(e3.at[0].at[pl.ds(sid * ER_T, ER_T)], eia_v)
    pltpu.sync_copy(e3.at[1].at[pl.ds(sid * ER_T, ER_T)], eib_v)
    for ei, hist, s0 in ((eia_v, hist_od, 0), (eib_v, hist_id, 4)):
      def sc(j, b):
        return pltpu.make_async_copy(ones_v, hist.at[ei.at[j]],
                                     hsem.at[s0 + b])
      for b in range(4):
        sc(b, b).start(add=True)
      @pl.loop(0, ER_T // 4 - 1)
      def _(j4):
        j = j4 * 4
        for b in range(4):
          sc(j + b, b).wait()
          sc(j + 4 + b, b).start(add=True)
      for b in range(4):
        sc(ER_T - 4 + b, b).wait()

  @pl.when(cid == 0)
  def _():
    hist_pass(e_d2s)

  @pl.when(cid == 1)
  def _():
    hist_pass(e_s2d)

  for d in emb:
    d.wait()
  for out, rows in ((hd_out, rows_d), (hs_out, rows_s)):
    pltpu.sync_copy(rows, out.at[pl.ds(wid * GPW, GPW)])

  plsc.subcore_barrier()

  def hist_out(out_od, out_id):
    for hist, out in ((hist_od, out_od), (hist_id, out_id)):
      pltpu.sync_copy(hist.at[pl.ds(sid * RPT, RPT)], hv_v)
      pltpu.sync_copy(hv_v, out.at[pl.ds(sid * RPT, RPT)])

  @pl.when(cid == 0)
  def _():
    hist_out(dg_od_d2s, dg_id_d2s)

  @pl.when(cid == 1)
  def _():
    hist_out(dg_od_s2d, dg_id_s2d)


_sc_front = functools.partial(
    pl.kernel,
    out_type=[jax.ShapeDtypeStruct((NP, H), f32),
              jax.ShapeDtypeStruct((NP, H), f32),
              jax.ShapeDtypeStruct((NP,), f32),
              jax.ShapeDtypeStruct((NP,), f32),
              jax.ShapeDtypeStruct((NP,), f32),
              jax.ShapeDtypeStruct((NP,), f32)],
    mesh=plsc.VectorSubcoreMesh(core_axis_name="c", subcore_axis_name="s",
                                num_cores=NC, num_subcores=NS),
    scratch_types=[pltpu.VMEM((4, 80), jnp.int32),
                   pltpu.VMEM((GPW, H), f32),
                   pltpu.VMEM((GPW, H), f32),
                   pltpu.VMEM((ER_T, 128), jnp.int32),
                   pltpu.VMEM((ER_T, 128), jnp.int32),
                   pltpu.VMEM((128,), f32),
                   pltpu.VMEM((RPT,), f32),
                   pltpu.VMEM_SHARED((NP,), f32),
                   pltpu.VMEM_SHARED((NP,), f32),
                   pltpu.SemaphoreType.DMA((4,)),
                   pltpu.SemaphoreType.DMA((8,))],
)(_sc_front_body)


NB = 4


def _sc_segsum_body(m_diag, m_desc, e_d2s, e_s2d,
                    agg_desc, agg_diag,
                    isrc_v, idst_v, bufs, z_v, acc, gsem, ssem):
  cid = lax.axis_index("c")
  sid = lax.axis_index("s")

  _fill128(z_v, 16, 0.0)
  @pl.loop(0, RPT // 16)
  def _(t):
    pltpu.sync_copy(z_v, acc.at[pl.ds(sid * RPT + t * 16, 16)])
  plsc.subcore_barrier()

  def edge_pass(m, e3):
    pltpu.sync_copy(e3.at[0].at[pl.ds(sid * ER_T, ER_T)], isrc_v)
    pltpu.sync_copy(e3.at[1].at[pl.ds(sid * ER_T, ER_T)], idst_v)

    def g(j, b):
      return pltpu.make_async_copy(m.at[isrc_v.at[j]], bufs.at[b],
                                   gsem.at[b])

    def s(j, b):
      return pltpu.make_async_copy(bufs.at[b], acc.at[idst_v.at[j]],
                                   ssem.at[b])

    g(0, 0).start()
    g(1, 1).start()
    g(0, 0).wait()
    s(0, 0).start(add=True)
    g(2, 2).start()
    g(1, 1).wait()
    s(1, 1).start(add=True)
    g(3, 3).start()

    @pl.loop(0, (ER_T - NB) // NB)
    def _(j4):
      j = j4 * NB + 2
      for i in range(NB):
        jj = j + i
        b = (2 + i) % NB
        s(jj - 2, (b - 2) % NB).wait()
        g(jj + 2, (b - 2) % NB).start()
        g(jj, b).wait()
        s(jj, b).start(add=True)

    jl = ER_T - 2
    s(jl - 2, (jl - 2) % NB).wait()
    s(jl - 1, (jl - 1) % NB).wait()
    g(jl, jl % NB).wait()
    s(jl, jl % NB).start(add=True)
    g(jl + 1, (jl + 1) % NB).wait()
    s(jl + 1, (jl + 1) % NB).start(add=True)
    s(jl, jl % NB).wait()
    s(jl + 1, (jl + 1) % NB).wait()

  @pl.when(cid == 0)
  def _():
    edge_pass(m_diag, e_d2s)

  @pl.when(cid == 1)
  def _():
    edge_pass(m_desc, e_s2d)

  plsc.subcore_barrier()

  def readout(out):
    @pl.loop(0, RPT // 16)
    def _(t):
      pltpu.sync_copy(acc.at[pl.ds(sid * RPT + t * 16, 16)], z_v)
      pltpu.sync_copy(z_v, out.at[pl.ds(sid * RPT + t * 16, 16)])

  @pl.when(cid == 0)
  def _():
    readout(agg_desc)

  @pl.when(cid == 1)
  def _():
    readout(agg_diag)


_sc_segsum = functools.partial(
    pl.kernel,
    out_type=[jax.ShapeDtypeStruct((NP, H), f32),
              jax.ShapeDtypeStruct((NP, H), f32)],
    mesh=plsc.VectorSubcoreMesh(core_axis_name="c", subcore_axis_name="s",
                                num_cores=NC, num_subcores=NS),
    scratch_types=[pltpu.VMEM((ER_T, 128), jnp.int32),
                   pltpu.VMEM((ER_T, 128), jnp.int32),
                   pltpu.VMEM((NB, 128, H), f32),
                   pltpu.VMEM((16, H), f32),
                   pltpu.VMEM_SHARED((NP, H), f32),
                   pltpu.SemaphoreType.DMA((NB,)),
                   pltpu.SemaphoreType.DMA((NB,))],
)(_sc_segsum_body)


def _rs(x):
  return lax.rsqrt(jnp.maximum(x, 1.0))


def _ln_scale_body(hd, hs, g, b, odd, ods, md, ms):
  def ln(x):
    mu = jnp.mean(x, axis=-1, keepdims=True)
    var = jnp.mean((x - mu) ** 2, axis=-1, keepdims=True)
    y = (x - mu) * lax.rsqrt(var + 1e-5) * g[...] + b[...]
    return jnp.nan_to_num(y, nan=0.0, posinf=0.0, neginf=0.0)
  md[...] = ln(hd[...]) * _rs(odd[...])
  ms[...] = ln(hs[...]) * _rs(ods[...])


def _ln_scale(hd, hs, g, b, odd, ods):
  row = pl.BlockSpec((BLK, H), lambda i: (i, 0))
  vec = pl.BlockSpec((1, H), lambda i: (0, 0))
  return pl.pallas_call(
      _ln_scale_body,
      grid=(RB,),
      in_specs=[row, row, vec, vec, row, row],
      out_specs=[row, row],
      out_shape=[jax.ShapeDtypeStruct((NP, H), f32)] * 2,
  )(hd, hs, g, b, odd, ods)


def _layer_body(aggd, aggg, idd, ids_, odd, ods, wd, bd, ws, bs, m1d, m1s):
  h1s = jnp.maximum(
      jnp.dot(aggd[...] * _rs(idd[...]), wd[...],
              preferred_element_type=f32) + bd[...], 0.0)
  m1s[...] = h1s * _rs(ods[...])
  h1d = jnp.maximum(
      jnp.dot(aggg[...] * _rs(ids_[...]), ws[...],
              preferred_element_type=f32) + bs[...], 0.0)
  m1d[...] = h1d * _rs(odd[...])


def _layer(aggd, aggg, idd, ids_, odd, ods, wd, bd, ws, bs):
  row = pl.BlockSpec((BLK, H), lambda i: (i, 0))
  mat = pl.BlockSpec((H, H), lambda i: (0, 0))
  vec = pl.BlockSpec((1, H), lambda i: (0, 0))
  return pl.pallas_call(
      _layer_body,
      grid=(RB,),
      in_specs=[row, row, row, row, row, row, mat, vec, mat, vec],
      out_specs=[row, row],
      out_shape=[jax.ShapeDtypeStruct((NP, H), f32)] * 2,
  )(aggd, aggg, idd, ids_, odd, ods, wd, bd, ws, bs)


def _pool_head_body(aggd, aggg, idd, ids_, wd, bd, ws, bs, pw, pb, out,
                    accd, accg):
  i = pl.program_id(0)
  h2s = jnp.maximum(
      jnp.dot(aggd[...] * _rs(idd[...]), wd[...],
              preferred_element_type=f32) + bd[...], 0.0)
  h2d = jnp.maximum(
      jnp.dot(aggg[...] * _rs(ids_[...]), ws[...],
              preferred_element_type=f32) + bs[...], 0.0)
  mask = (lax.broadcasted_iota(jnp.int32, (BLK, H), 0) + i * BLK) < N_NODE
  sd = jnp.sum(jnp.where(mask, h2s, 0.0), axis=0, keepdims=True)
  sg = jnp.sum(jnp.where(mask, h2d, 0.0), axis=0, keepdims=True)

  @pl.when(i == 0)
  def _():
    accd[...] = sd
    accg[...] = sg

  @pl.when(i > 0)
  def _():
    accd[...] += sd
    accg[...] += sg

  @pl.when(i == RB - 1)
  def _():
    g = (accd[...] + accg[...]) / float(N_NODE)
    g = jnp.nan_to_num(g, nan=0.0, posinf=0.0, neginf=0.0)
    o = jnp.dot(g, pw[...], preferred_element_type=f32) + pb[...]
    o = jnp.nan_to_num(o, nan=0.0, posinf=0.0, neginf=0.0)
    n = jnp.sqrt(jnp.sum(o * o, axis=-1, keepdims=True))
    out[...] = o / jnp.maximum(n, 1e-6)


def _pool_head(aggd, aggg, idd, ids_, wd, bd, ws, bs, pw, pb):
  row = pl.BlockSpec((BLK, H), lambda i: (i, 0))
  mat = pl.BlockSpec((H, H), lambda i: (0, 0))
  vec = pl.BlockSpec((1, H), lambda i: (0, 0))
  return pl.pallas_call(
      _pool_head_body,
      grid=(RB,),
      in_specs=[row, row, row, row, mat, vec, mat, vec,
                pl.BlockSpec((H, OUT_DIM), lambda i: (0, 0)),
                pl.BlockSpec((1, OUT_DIM), lambda i: (0, 0))],
      out_specs=pl.BlockSpec((1, OUT_DIM), lambda i: (0, 0)),
      out_shape=jax.ShapeDtypeStruct((1, OUT_DIM), f32),
      scratch_shapes=[pltpu.VMEM((1, H), f32), pltpu.VMEM((1, H), f32)],
  )(aggd, aggg, idd, ids_, wd, bd, ws, bs, pw, pb)


def _pad_edges(ei):
  pad = N_NODE + (jnp.arange(EP - E, dtype=jnp.int32) % (NP - N_NODE))
  return jnp.concatenate(
      [ei, jnp.broadcast_to(pad, (2, EP - E))], axis=1).reshape(2, ER, 128)


def kernel(diag_tid, desc_tid, ei_d2s, ei_s2d, diag_table, desc_table,
           ln_g, ln_b, W_d2s_0, b_d2s_0, W_s2d_0, b_s2d_0,
           W_d2s_1, b_d2s_1, W_s2d_1, b_s2d_1, proj_W, proj_b):
  pad_tid = jnp.arange(NP - N_NODE, dtype=jnp.int32)
  diag_tid_p = jnp.concatenate([diag_tid, pad_tid])
  desc_tid_p = jnp.concatenate([desc_tid, pad_tid])
  e_d2s = _pad_edges(ei_d2s)
  e_s2d = _pad_edges(ei_s2d)

  hd_raw, hs_raw, dg_od_d2s, dg_id_d2s, dg_od_s2d, dg_id_s2d = _sc_front(
      diag_tid_p, desc_tid_p, e_d2s, e_s2d,
      diag_table, desc_table,
      jnp.ones((128,), f32), jnp.zeros((RPT,), f32))

  def bcast(dg):
    return jnp.broadcast_to(dg[:, None], (NP, H))

  odb_d2s, idb_d2s = bcast(dg_od_d2s), bcast(dg_id_d2s)
  odb_s2d, idb_s2d = bcast(dg_od_s2d), bcast(dg_id_s2d)

  g2, b2 = ln_g.reshape(1, H), ln_b.reshape(1, H)
  m0_diag, m0_desc = _ln_scale(hd_raw, hs_raw, g2, b2, odb_d2s, odb_s2d)

  agg0_desc, agg0_diag = _sc_segsum(m0_diag, m0_desc, e_d2s, e_s2d)

  m1_diag, m1_desc = _layer(agg0_desc, agg0_diag, idb_d2s, idb_s2d,
                            odb_d2s, odb_s2d,
                            W_d2s_0, b_d2s_0.reshape(1, H),
                            W_s2d_0, b_s2d_0.reshape(1, H))

  agg1_desc, agg1_diag = _sc_segsum(m1_diag, m1_desc, e_d2s, e_s2d)

  return _pool_head(agg1_desc, agg1_diag, idb_d2s, idb_s2d,
                    W_d2s_1, b_d2s_1.reshape(1, H),
                    W_s2d_1, b_s2d_1.reshape(1, H),
                    proj_W, proj_b.reshape(1, OUT_DIM))
